# baseline (speedup 1.0000x reference)
import functools

import jax
import jax.numpy as jnp
from jax.experimental import pallas as pl
from jax.experimental.pallas import tpu as pltpu
from jax.sharding import Mesh, PartitionSpec as P


def _gtu_body(x_hbm, w_ref, b_ref, o_hbm, xbuf, obuf, in_sems, out_sems,
              *, B, C, K, N, TN, T, T_out, NC):
    L = TN * T
    CS = C // NC
    BLK_PER_B = N // TN
    NT = B * BLK_PER_B
    xf = x_hbm.reshape(B, C, N * T)

    def start_in(slot, t):
        b = t // BLK_PER_B
        j = t % BLK_PER_B
        for i in range(NC):
            pltpu.make_async_copy(
                xf.at[b, pl.ds(i * CS, CS), pl.ds(j * L, L)],
                xbuf.at[slot, pl.ds(i * CS, CS)],
                in_sems.at[slot, i]).start()

    def wait_in(slot):
        for i in range(NC):
            pltpu.make_async_copy(
                xf.at[0, pl.ds(i * CS, CS), pl.ds(0, L)],
                xbuf.at[slot, pl.ds(i * CS, CS)],
                in_sems.at[slot, i]).wait()

    def start_out(slot, t):
        b = t // BLK_PER_B
        j = t % BLK_PER_B
        for i in range(NC):
            pltpu.make_async_copy(
                obuf.at[slot, pl.ds(i * CS, CS)],
                o_hbm.at[b, pl.ds(i * CS, CS), pl.ds(j * TN, TN), :],
                out_sems.at[slot, i]).start()

    def wait_out(slot):
        for i in range(NC):
            pltpu.make_async_copy(
                obuf.at[slot, pl.ds(i * CS, CS)],
                o_hbm.at[0, pl.ds(i * CS, CS), pl.ds(0, TN), :],
                out_sems.at[slot, i]).wait()

    def compute(slot):
        xb = xbuf[slot].astype(jnp.bfloat16)
        parts = [xb] + [jnp.roll(xb, -k, axis=1) for k in range(1, K)]
        xp = jnp.concatenate(parts, axis=0)
        r = jnp.dot(w_ref[...], xp,
                    preferred_element_type=jnp.float32)
        r = r + b_ref[...]
        g = jnp.tanh(r[:C]) * (0.5 + 0.5 * jnp.tanh(r[C:]))
        obuf[slot] = g.reshape(C, TN, T)[:, :, :T_out]

    start_in(0, 0)

    def body(k, carry):
        slot = jax.lax.rem(k, 2)
        nslot = jax.lax.rem(k + 1, 2)

        @pl.when(k + 1 < NT)
        def _():
            start_in(nslot, k + 1)

        wait_in(slot)

        @pl.when(k >= 2)
        def _():
            wait_out(slot)

        compute(slot)
        start_out(slot, k)
        return carry

    jax.lax.fori_loop(0, NT, body, 0)
    wait_out(jax.lax.rem(NT - 2, 2))
    wait_out(jax.lax.rem(NT - 1, 2))


def _gtu_call(x, w2, b2, *, C, K, N, T, TN, T_out, NC):
    B = x.shape[0]
    C2 = 2 * C
    body = functools.partial(_gtu_body, B=B, C=C, K=K, N=N, TN=TN, T=T,
                             T_out=T_out, NC=NC)
    return pl.pallas_call(
        body,
        out_shape=jax.ShapeDtypeStruct((B, C, N, T_out), jnp.float32),
        grid=(1,),
        in_specs=[
            pl.BlockSpec(memory_space=pl.ANY),
            pl.BlockSpec((C2, K * C), lambda p: (0, 0)),
            pl.BlockSpec((C2, 1), lambda p: (0, 0)),
        ],
        out_specs=pl.BlockSpec(memory_space=pl.ANY),
        scratch_shapes=[
            pltpu.VMEM((2, C, TN * T), jnp.float32),
            pltpu.VMEM((2, C, TN, T_out), jnp.float32),
            pltpu.SemaphoreType.DMA((2, NC)),
            pltpu.SemaphoreType.DMA((2, NC)),
        ],
    )(x, w2, b2)


def kernel(x, weight, bias):
    B, C, N, T = x.shape
    K = weight.shape[-1]
    C2 = 2 * C
    T_out = T - K + 1

    TN = 128
    NC = 4

    w2 = jnp.transpose(weight[:, :, 0, :], (0, 2, 1)).reshape(C2, K * C)
    scale = jnp.concatenate([jnp.ones((C, 1)), jnp.full((C, 1), 0.5)], axis=0)
    w2 = (w2 * scale).astype(jnp.bfloat16)
    b2 = bias.reshape(C2, 1) * scale

    f = functools.partial(_gtu_call, C=C, K=K, N=N, T=T, TN=TN,
                          T_out=T_out, NC=NC)

    devs = jax.devices()
    if len(devs) >= 2 and B % 2 == 0:
        mesh = Mesh(devs[:2], ("d",))
        f = jax.shard_map(f, mesh=mesh,
                          in_specs=(P("d"), P(), P()),
                          out_specs=P("d"), check_vma=False)
    return f(x, w2, b2)


# speedup vs baseline: 2.6188x; 2.6188x over previous
import functools

import jax
import jax.numpy as jnp
from jax.experimental import pallas as pl
from jax.experimental.pallas import tpu as pltpu

NSLOT = 3


def _gtu_body(x_hbm, w_ref, b_ref, o_hbm, xbuf, obuf, in_sems, out_sems,
              *, B, C, K, N, TN, T, T_out, NCI, NCO):
    L = TN * T
    CSI = C // NCI
    CSO = C // NCO
    BLK_PER_B = N // TN
    NT = B * BLK_PER_B
    xf = x_hbm.reshape(B, C, N * T)

    def start_in(slot, t):
        b = t // BLK_PER_B
        j = t % BLK_PER_B
        for i in range(NCI):
            pltpu.make_async_copy(
                xf.at[b, pl.ds(i * CSI, CSI), pl.ds(j * L, L)],
                xbuf.at[slot, pl.ds(i * CSI, CSI)],
                in_sems.at[slot, i]).start()

    def wait_in(slot):
        for i in range(NCI):
            pltpu.make_async_copy(
                xf.at[0, pl.ds(i * CSI, CSI), pl.ds(0, L)],
                xbuf.at[slot, pl.ds(i * CSI, CSI)],
                in_sems.at[slot, i]).wait()

    def start_out(slot, t):
        b = t // BLK_PER_B
        j = t % BLK_PER_B
        for i in range(NCO):
            pltpu.make_async_copy(
                obuf.at[slot, pl.ds(i * CSO, CSO)],
                o_hbm.at[b, pl.ds(i * CSO, CSO), pl.ds(j * TN, TN), :],
                out_sems.at[slot, i]).start()

    def wait_out(slot):
        for i in range(NCO):
            pltpu.make_async_copy(
                obuf.at[slot, pl.ds(i * CSO, CSO)],
                o_hbm.at[0, pl.ds(i * CSO, CSO), pl.ds(0, TN), :],
                out_sems.at[slot, i]).wait()

    def compute(slot):
        xb = xbuf[slot].astype(jnp.bfloat16)
        parts = [xb] + [jnp.roll(xb, -k, axis=1) for k in range(1, K)]
        xp = jnp.concatenate(parts, axis=0)
        r = jnp.dot(w_ref[...], xp,
                    preferred_element_type=jnp.float32)
        r = r + b_ref[...]
        g = jnp.tanh(r[:C]) * (0.5 + 0.5 * jnp.tanh(r[C:]))
        obuf[slot] = g.reshape(C, TN, T)[:, :, :T_out]

    for s in range(NSLOT - 1):
        start_in(s, s)

    def body(k, carry):
        slot = jax.lax.rem(k, NSLOT)
        nslot = jax.lax.rem(k + NSLOT - 1, NSLOT)

        @pl.when(k + NSLOT - 1 < NT)
        def _():
            start_in(nslot, k + NSLOT - 1)

        wait_in(slot)

        @pl.when(k >= NSLOT)
        def _():
            wait_out(slot)

        compute(slot)
        start_out(slot, k)
        return carry

    jax.lax.fori_loop(0, NT, body, 0)
    for s in range(NSLOT):
        wait_out(jax.lax.rem(NT - NSLOT + s, NSLOT))


def kernel(x, weight, bias):
    B, C, N, T = x.shape
    K = weight.shape[-1]
    C2 = 2 * C
    T_out = T - K + 1

    TN = 128
    NCI = 4
    NCO = 4

    w2 = jnp.transpose(weight[:, :, 0, :], (0, 2, 1)).reshape(C2, K * C)
    scale = jnp.concatenate([jnp.ones((C, 1)), jnp.full((C, 1), 0.5)], axis=0)
    w2 = (w2 * scale).astype(jnp.bfloat16)
    b2 = bias.reshape(C2, 1) * scale

    body = functools.partial(_gtu_body, B=B, C=C, K=K, N=N, TN=TN, T=T,
                             T_out=T_out, NCI=NCI, NCO=NCO)
    return pl.pallas_call(
        body,
        out_shape=jax.ShapeDtypeStruct((B, C, N, T_out), jnp.float32),
        grid=(1,),
        in_specs=[
            pl.BlockSpec(memory_space=pl.ANY),
            pl.BlockSpec((C2, K * C), lambda p: (0, 0)),
            pl.BlockSpec((C2, 1), lambda p: (0, 0)),
        ],
        out_specs=pl.BlockSpec(memory_space=pl.ANY),
        scratch_shapes=[
            pltpu.VMEM((NSLOT, C, TN * T), jnp.float32),
            pltpu.VMEM((NSLOT, C, TN, T_out), jnp.float32),
            pltpu.SemaphoreType.DMA((NSLOT, NCI)),
            pltpu.SemaphoreType.DMA((NSLOT, NCO)),
        ],
    )(x, w2, b2)
